# Initial kernel scaffold; baseline (speedup 1.0000x reference)
import jax
import jax.numpy as jnp
from jax.experimental import pallas as pl


def kernel(x, W, b):
    raise NotImplementedError("write your pallas kernel here")


# speedup vs baseline: 1.3141x; 1.3141x over previous
import jax
import jax.numpy as jnp
from jax import lax
from jax.experimental import pallas as pl

HIDDEN = 768
EXPERTS = 64
KTOP = 2
EPS = 1e-06
TOKENS = 32768
BLOCK_T = 512


def _router_block(x_ref, wt_ref, b_ref, lg_ref, w_ref, i_ref, m_ref):
    lg = jnp.dot(x_ref[...], wt_ref[...], preferred_element_type=jnp.float32)
    lg = lg + b_ref[...]
    lg_ref[...] = lg

    mx = jnp.max(lg, axis=-1, keepdims=True)
    ssum = jnp.sum(jnp.exp(lg - mx), axis=-1, keepdims=True)

    iota = lax.broadcasted_iota(jnp.int32, lg.shape, 1)
    big = jnp.int32(EXPERTS)
    i1 = jnp.min(jnp.where(lg == mx, iota, big), axis=-1, keepdims=True)
    mask1 = iota == i1
    l2 = jnp.max(jnp.where(mask1, -jnp.inf, lg), axis=-1, keepdims=True)
    i2 = jnp.min(jnp.where((lg == l2) & (~mask1), iota, big), axis=-1, keepdims=True)
    mask2 = iota == i2

    e2 = jnp.exp(l2 - mx)
    inv = 1.0 / (1.0 + e2 + EPS * ssum)
    w_ref[...] = jnp.concatenate([inv, e2 * inv], axis=1)
    i_ref[...] = jnp.concatenate([i1, i2], axis=1)
    m_ref[...] = (mask1 | mask2).astype(jnp.float32)


def kernel(x, W, b):
    wt = W.T
    b2 = b.reshape(1, EXPERTS)
    grid = (TOKENS // BLOCK_T,)
    out = pl.pallas_call(
        _router_block,
        grid=grid,
        in_specs=[
            pl.BlockSpec((BLOCK_T, HIDDEN), lambda i: (i, 0)),
            pl.BlockSpec((HIDDEN, EXPERTS), lambda i: (0, 0)),
            pl.BlockSpec((1, EXPERTS), lambda i: (0, 0)),
        ],
        out_specs=[
            pl.BlockSpec((BLOCK_T, EXPERTS), lambda i: (i, 0)),
            pl.BlockSpec((BLOCK_T, KTOP), lambda i: (i, 0)),
            pl.BlockSpec((BLOCK_T, KTOP), lambda i: (i, 0)),
            pl.BlockSpec((BLOCK_T, EXPERTS), lambda i: (i, 0)),
        ],
        out_shape=[
            jax.ShapeDtypeStruct((TOKENS, EXPERTS), jnp.float32),
            jax.ShapeDtypeStruct((TOKENS, KTOP), jnp.float32),
            jax.ShapeDtypeStruct((TOKENS, KTOP), jnp.int32),
            jax.ShapeDtypeStruct((TOKENS, EXPERTS), jnp.float32),
        ],
    )(x, wt, b2)
    return tuple(out)


# speedup vs baseline: 1.5683x; 1.1935x over previous
import jax
import jax.numpy as jnp
from jax import lax
from jax.experimental import pallas as pl

HIDDEN = 768
EXPERTS = 64
KTOP = 2
EPS = 1e-06
TOKENS = 32768
BLOCK_T = 1024


def _router_block(x_ref, wt_ref, b_ref, lg_ref, w_ref, i_ref, m_ref):
    lg = jnp.dot(x_ref[...], wt_ref[...], preferred_element_type=jnp.float32)
    lg = lg + b_ref[...]
    lg_ref[...] = lg

    mx = jnp.max(lg, axis=-1, keepdims=True)
    ssum = jnp.sum(jnp.exp(lg - mx), axis=-1, keepdims=True)

    iota = lax.broadcasted_iota(jnp.int32, lg.shape, 1)
    big = jnp.int32(EXPERTS)
    i1 = jnp.min(jnp.where(lg == mx, iota, big), axis=-1, keepdims=True)
    mask1 = iota == i1
    l2 = jnp.max(jnp.where(mask1, -jnp.inf, lg), axis=-1, keepdims=True)
    i2 = jnp.min(jnp.where((lg == l2) & (~mask1), iota, big), axis=-1, keepdims=True)
    mask2 = iota == i2

    e2 = jnp.exp(l2 - mx)
    inv = 1.0 / (1.0 + e2 + EPS * ssum)
    w_ref[...] = jnp.concatenate([inv, e2 * inv], axis=1)
    i_ref[...] = jnp.concatenate([i1, i2], axis=1)
    m_ref[...] = (mask1 | mask2).astype(jnp.float32)


def kernel(x, W, b):
    wt = W.T
    b2 = b.reshape(1, EXPERTS)
    grid = (TOKENS // BLOCK_T,)
    out = pl.pallas_call(
        _router_block,
        grid=grid,
        in_specs=[
            pl.BlockSpec((BLOCK_T, HIDDEN), lambda i: (i, 0)),
            pl.BlockSpec((HIDDEN, EXPERTS), lambda i: (0, 0)),
            pl.BlockSpec((1, EXPERTS), lambda i: (0, 0)),
        ],
        out_specs=[
            pl.BlockSpec((BLOCK_T, EXPERTS), lambda i: (i, 0)),
            pl.BlockSpec((BLOCK_T, KTOP), lambda i: (i, 0)),
            pl.BlockSpec((BLOCK_T, KTOP), lambda i: (i, 0)),
            pl.BlockSpec((BLOCK_T, EXPERTS), lambda i: (i, 0)),
        ],
        out_shape=[
            jax.ShapeDtypeStruct((TOKENS, EXPERTS), jnp.float32),
            jax.ShapeDtypeStruct((TOKENS, KTOP), jnp.float32),
            jax.ShapeDtypeStruct((TOKENS, KTOP), jnp.int32),
            jax.ShapeDtypeStruct((TOKENS, EXPERTS), jnp.float32),
        ],
    )(x, wt, b2)
    return tuple(out)


# speedup vs baseline: 1.7400x; 1.1095x over previous
import jax
import jax.numpy as jnp
from jax import lax
from jax.experimental import pallas as pl

HIDDEN = 768
EXPERTS = 64
KTOP = 2
EPS = 1e-06
TOKENS = 32768
BLOCK_T = 2048


def _router_block(x_ref, wt_ref, b_ref, lg_ref, w_ref, i_ref, m_ref):
    lg = jnp.dot(x_ref[...], wt_ref[...], preferred_element_type=jnp.float32)
    lg = lg + b_ref[...]
    lg_ref[...] = lg

    mx = jnp.max(lg, axis=-1, keepdims=True)
    ssum = jnp.sum(jnp.exp(lg - mx), axis=-1, keepdims=True)

    iota = lax.broadcasted_iota(jnp.int32, lg.shape, 1)
    big = jnp.int32(EXPERTS)
    i1 = jnp.min(jnp.where(lg == mx, iota, big), axis=-1, keepdims=True)
    mask1 = iota == i1
    l2 = jnp.max(jnp.where(mask1, -jnp.inf, lg), axis=-1, keepdims=True)
    i2 = jnp.min(jnp.where((lg == l2) & (~mask1), iota, big), axis=-1, keepdims=True)
    mask2 = iota == i2

    e2 = jnp.exp(l2 - mx)
    inv = 1.0 / (1.0 + e2 + EPS * ssum)
    w_ref[...] = jnp.concatenate([inv, e2 * inv], axis=1)
    i_ref[...] = jnp.concatenate([i1, i2], axis=1)
    m_ref[...] = (mask1 | mask2).astype(jnp.float32)


def kernel(x, W, b):
    wt = W.T
    b2 = b.reshape(1, EXPERTS)
    grid = (TOKENS // BLOCK_T,)
    out = pl.pallas_call(
        _router_block,
        grid=grid,
        in_specs=[
            pl.BlockSpec((BLOCK_T, HIDDEN), lambda i: (i, 0)),
            pl.BlockSpec((HIDDEN, EXPERTS), lambda i: (0, 0)),
            pl.BlockSpec((1, EXPERTS), lambda i: (0, 0)),
        ],
        out_specs=[
            pl.BlockSpec((BLOCK_T, EXPERTS), lambda i: (i, 0)),
            pl.BlockSpec((BLOCK_T, KTOP), lambda i: (i, 0)),
            pl.BlockSpec((BLOCK_T, KTOP), lambda i: (i, 0)),
            pl.BlockSpec((BLOCK_T, EXPERTS), lambda i: (i, 0)),
        ],
        out_shape=[
            jax.ShapeDtypeStruct((TOKENS, EXPERTS), jnp.float32),
            jax.ShapeDtypeStruct((TOKENS, KTOP), jnp.float32),
            jax.ShapeDtypeStruct((TOKENS, KTOP), jnp.int32),
            jax.ShapeDtypeStruct((TOKENS, EXPERTS), jnp.float32),
        ],
    )(x, wt, b2)
    return tuple(out)


# speedup vs baseline: 1.7867x; 1.0268x over previous
import jax
import jax.numpy as jnp
from jax import lax
from jax.experimental import pallas as pl

HIDDEN = 768
EXPERTS = 64
KTOP = 2
EPS = 1e-06
TOKENS = 32768
BLOCK_T = 4096


def _router_block(x_ref, wt_ref, b_ref, lg_ref, w_ref, i_ref, m_ref):
    lg = jnp.dot(x_ref[...], wt_ref[...], preferred_element_type=jnp.float32)
    lg = lg + b_ref[...]
    lg_ref[...] = lg

    mx = jnp.max(lg, axis=-1, keepdims=True)
    ssum = jnp.sum(jnp.exp(lg - mx), axis=-1, keepdims=True)

    iota = lax.broadcasted_iota(jnp.int32, lg.shape, 1)
    big = jnp.int32(EXPERTS)
    i1 = jnp.min(jnp.where(lg == mx, iota, big), axis=-1, keepdims=True)
    mask1 = iota == i1
    l2 = jnp.max(jnp.where(mask1, -jnp.inf, lg), axis=-1, keepdims=True)
    i2 = jnp.min(jnp.where((lg == l2) & (~mask1), iota, big), axis=-1, keepdims=True)
    mask2 = iota == i2

    e2 = jnp.exp(l2 - mx)
    inv = 1.0 / (1.0 + e2 + EPS * ssum)
    w_ref[...] = jnp.concatenate([inv, e2 * inv], axis=1)
    i_ref[...] = jnp.concatenate([i1, i2], axis=1)
    m_ref[...] = (mask1 | mask2).astype(jnp.float32)


def kernel(x, W, b):
    wt = W.T
    b2 = b.reshape(1, EXPERTS)
    grid = (TOKENS // BLOCK_T,)
    out = pl.pallas_call(
        _router_block,
        grid=grid,
        in_specs=[
            pl.BlockSpec((BLOCK_T, HIDDEN), lambda i: (i, 0)),
            pl.BlockSpec((HIDDEN, EXPERTS), lambda i: (0, 0)),
            pl.BlockSpec((1, EXPERTS), lambda i: (0, 0)),
        ],
        out_specs=[
            pl.BlockSpec((BLOCK_T, EXPERTS), lambda i: (i, 0)),
            pl.BlockSpec((BLOCK_T, KTOP), lambda i: (i, 0)),
            pl.BlockSpec((BLOCK_T, KTOP), lambda i: (i, 0)),
            pl.BlockSpec((BLOCK_T, EXPERTS), lambda i: (i, 0)),
        ],
        out_shape=[
            jax.ShapeDtypeStruct((TOKENS, EXPERTS), jnp.float32),
            jax.ShapeDtypeStruct((TOKENS, KTOP), jnp.float32),
            jax.ShapeDtypeStruct((TOKENS, KTOP), jnp.int32),
            jax.ShapeDtypeStruct((TOKENS, EXPERTS), jnp.float32),
        ],
    )(x, wt, b2)
    return tuple(out)
